# Initial kernel scaffold; baseline (speedup 1.0000x reference)
import functools

import jax
import jax.numpy as jnp
from jax import lax
from jax.experimental import pallas as pl
from jax.experimental.pallas import tpu as pltpu
from jax.experimental.pallas import tpu_sc as plsc

N = 10000
E = 320000
D = 128
H = 16
C = 16

NC = 2
NS = 16
NW = NC * NS
L = 16

NPAD = 10240
NPT = NPAD // NS

CHUNK = 128
NCHUNKS = E // CHUNK
KMAX = -(-NCHUNKS // NW)

DCH = 2000
EPW = E // NW

_MESH = dict(core_axis_name="c", subcore_axis_name="s")


def _sc_degree(dst):
    @functools.partial(
        pl.kernel,
        out_type=jax.ShapeDtypeStruct((NC, NPAD), jnp.float32),
        mesh=plsc.VectorSubcoreMesh(**_MESH),
        scratch_types=[
            pltpu.VMEM((NPAD,), jnp.float32),
            pltpu.VMEM((DCH,), jnp.int32),
            pltpu.VMEM((NS, NPT), jnp.float32),
            pltpu.VMEM((NPT,), jnp.float32),
            pltpu.VMEM_SHARED((NS, NPAD), jnp.float32),
        ],
    )
    def k(dst_hbm, out_hbm, hist, dbuf, redbuf, redout, stage):
        cid = lax.axis_index("c")
        sid = lax.axis_index("s")
        wid = sid * NC + cid

        @pl.loop(0, NPAD // L)
        def _(i):
            hist[pl.ds(i * L, L)] = jnp.zeros((L,), jnp.float32)

        base = wid * EPW

        @pl.loop(0, EPW // DCH)
        def _(c0):
            pltpu.sync_copy(dst_hbm.at[pl.ds(base + c0 * DCH, DCH)], dbuf)

            @pl.loop(0, DCH // L)
            def _(j):
                idx = dbuf[pl.ds(j * L, L)]
                plsc.addupdate_scatter(hist, [idx], jnp.ones((L,), jnp.float32))

        pltpu.sync_copy(hist, stage.at[sid])
        plsc.subcore_barrier()
        for r in range(NS):
            pltpu.sync_copy(stage.at[r, pl.ds(sid * NPT, NPT)], redbuf.at[r])

        @pl.loop(0, NPT // L)
        def _(i):
            v = redbuf[0, pl.ds(i * L, L)]
            for r in range(1, NS):
                v = v + redbuf[r, pl.ds(i * L, L)]
            redout[pl.ds(i * L, L)] = v

        pltpu.sync_copy(redout, out_hbm.at[cid, pl.ds(sid * NPT, NPT)])

    return k(dst)


def _sc_scatter(src, dst, vals):
    @functools.partial(
        pl.kernel,
        out_type=jax.ShapeDtypeStruct((NC, NPAD, H), jnp.float32),
        mesh=plsc.VectorSubcoreMesh(**_MESH),
        scratch_types=[
            pltpu.VMEM((CHUNK,), jnp.int32),
            pltpu.VMEM((CHUNK,), jnp.int32),
            pltpu.VMEM((CHUNK, H), jnp.float32),
            pltpu.VMEM((NPT, H), jnp.float32),
            pltpu.VMEM_SHARED((NPAD, H), jnp.float32),
            pltpu.SemaphoreType.DMA,
        ],
    )
    def k(src_hbm, dst_hbm, vals_hbm, out_hbm, sidx, didx, rows, zbuf, acc, sem):
        cid = lax.axis_index("c")
        sid = lax.axis_index("s")
        wid = sid * NC + cid

        @pl.loop(0, NPT)
        def _(i):
            zbuf[i, :] = jnp.zeros((H,), jnp.float32)

        pltpu.sync_copy(zbuf, acc.at[pl.ds(sid * NPT, NPT)])
        plsc.subcore_barrier()

        @pl.loop(0, KMAX)
        def _(kk):
            c0 = wid + kk * NW

            @pl.when(c0 < NCHUNKS)
            def _():
                base = c0 * CHUNK
                pltpu.sync_copy(src_hbm.at[pl.ds(base, CHUNK)], sidx)
                pltpu.sync_copy(dst_hbm.at[pl.ds(base, CHUNK)], didx)
                pltpu.async_copy(vals_hbm.at[sidx], rows, sem).wait()
                pltpu.sync_copy(rows, acc.at[didx], add=True)

        plsc.subcore_barrier()
        pltpu.sync_copy(
            acc.at[pl.ds(sid * NPT, NPT)],
            out_hbm.at[cid, pl.ds(sid * NPT, NPT)],
        )

    return k(src, dst, vals)


def _tc_matmul1(x, W1):
    def body(x_ref, w_ref, o_ref):
        o_ref[...] = jnp.dot(
            x_ref[...], w_ref[...], preferred_element_type=jnp.float32
        )

    return pl.pallas_call(
        body, out_shape=jax.ShapeDtypeStruct((N, H), jnp.float32)
    )(x, W1)


def _tc_scale(h, degp_t):
    def body(h_ref, d_ref, hs_ref, dis_ref):
        deg = d_ref[:N, 0:1] + d_ref[:N, 1:2] + 1.0
        dis = lax.rsqrt(deg)
        dis_ref[...] = dis
        hs_ref[...] = h_ref[...] * dis

    return pl.pallas_call(
        body,
        out_shape=(
            jax.ShapeDtypeStruct((N, H), jnp.float32),
            jax.ShapeDtypeStruct((N, 1), jnp.float32),
        ),
    )(h, degp_t)


def _tc_mid(acc, hs, dis, W2, b1):
    def body(a_ref, hs_ref, dis_ref, w_ref, b_ref, o_ref):
        s = a_ref[0, :N, :] + a_ref[1, :N, :] + hs_ref[...]
        out1 = jnp.maximum(s * dis_ref[...] + b_ref[...], 0.0)
        o_ref[...] = (
            jnp.dot(out1, w_ref[...], preferred_element_type=jnp.float32)
            * dis_ref[...]
        )

    return pl.pallas_call(
        body, out_shape=jax.ShapeDtypeStruct((N, C), jnp.float32)
    )(acc, hs, dis, W2, b1)


def _tc_final(acc, h2s, dis, b2):
    def body(a_ref, hs_ref, dis_ref, b_ref, o_ref):
        s = a_ref[0, :N, :] + a_ref[1, :N, :] + hs_ref[...]
        o = s * dis_ref[...] + b_ref[...]
        m = jnp.max(o, axis=1, keepdims=True)
        lse = jnp.log(jnp.sum(jnp.exp(o - m), axis=1, keepdims=True)) + m
        o_ref[...] = o - lse

    return pl.pallas_call(
        body, out_shape=jax.ShapeDtypeStruct((N, C), jnp.float32)
    )(acc, h2s, dis, b2)


def kernel(x, edge_index, W1, b1, W2, b2):
    src = edge_index[0].astype(jnp.int32)
    dst = edge_index[1].astype(jnp.int32)
    degp = _sc_degree(dst)
    h = _tc_matmul1(x, W1)
    hs, dis = _tc_scale(h, degp.T)
    acc1 = _sc_scatter(src, dst, hs)
    h2s = _tc_mid(acc1, hs, dis, W2, b1.reshape(1, H))
    acc2 = _sc_scatter(src, dst, h2s)
    return _tc_final(acc2, h2s, dis, b2.reshape(1, C))


# speedup vs baseline: 23.4919x; 23.4919x over previous
import dataclasses
import functools

import jax
import jax.numpy as jnp
from jax import lax
from jax.experimental import pallas as pl
from jax.experimental.pallas import tpu as pltpu
from jax.experimental.pallas import tpu_sc as plsc

N = 10000
E = 320000
D = 128
H = 16
C = 16

NC = 2
NS = 16
NW = NC * NS
L = 16

NPAD = 10240
NPT = NPAD // NS

CHUNK = 128
NCHUNKS = E // CHUNK
KMAX = -(-NCHUNKS // NW)

DCH = 2000
EPW = E // NW

_MESH = dict(core_axis_name="c", subcore_axis_name="s")

_SC_PARAMS = pltpu.CompilerParams()
if "needs_layout_passes" in pltpu.CompilerParams.__dataclass_fields__:
    _SC_PARAMS = dataclasses.replace(
        _SC_PARAMS, needs_layout_passes=False, use_tc_tiling_on_sc=False
    )


def _sc_degree(dst):
    @functools.partial(
        pl.kernel,
        out_type=jax.ShapeDtypeStruct((NC, NPAD), jnp.float32),
        mesh=plsc.VectorSubcoreMesh(**_MESH),
        compiler_params=_SC_PARAMS,
        scratch_types=[
            pltpu.VMEM((NPAD,), jnp.float32),
            pltpu.VMEM((DCH,), jnp.int32),
            pltpu.VMEM((NS, NPT), jnp.float32),
            pltpu.VMEM((NPT,), jnp.float32),
            pltpu.VMEM_SHARED((NS, NPAD), jnp.float32),
        ],
    )
    def k(dst_hbm, out_hbm, hist, dbuf, redbuf, redout, stage):
        cid = lax.axis_index("c")
        sid = lax.axis_index("s")
        wid = sid * NC + cid

        @pl.loop(0, NPAD // L)
        def _(i):
            hist[pl.ds(i * L, L)] = jnp.zeros((L,), jnp.float32)

        base = wid * EPW

        @pl.loop(0, EPW // DCH)
        def _(c0):
            pltpu.sync_copy(dst_hbm.at[pl.ds(base + c0 * DCH, DCH)], dbuf)

            @pl.loop(0, DCH // L)
            def _(j):
                idx = dbuf[pl.ds(j * L, L)]
                plsc.addupdate_scatter(hist, [idx], jnp.ones((L,), jnp.float32))

        pltpu.sync_copy(hist, stage.at[sid])
        plsc.subcore_barrier()
        for r in range(NS):
            pltpu.sync_copy(stage.at[r, pl.ds(sid * NPT, NPT)], redbuf.at[r])

        @pl.loop(0, NPT // L)
        def _(i):
            v = redbuf[0, pl.ds(i * L, L)]
            for r in range(1, NS):
                v = v + redbuf[r, pl.ds(i * L, L)]
            redout[pl.ds(i * L, L)] = v

        pltpu.sync_copy(redout, out_hbm.at[cid, pl.ds(sid * NPT, NPT)])

    return k(dst)


def _sc_scatter(src, dst, vals):
    @functools.partial(
        pl.kernel,
        out_type=jax.ShapeDtypeStruct((NC, NPAD, H), jnp.float32),
        mesh=plsc.VectorSubcoreMesh(**_MESH),
        compiler_params=_SC_PARAMS,
        scratch_types=[
            pltpu.VMEM((CHUNK,), jnp.int32),
            pltpu.VMEM((CHUNK,), jnp.int32),
            pltpu.VMEM((CHUNK, H), jnp.float32),
            pltpu.VMEM((NPT, H), jnp.float32),
            pltpu.VMEM_SHARED((NPAD, H), jnp.float32),
            pltpu.SemaphoreType.DMA,
        ],
    )
    def k(src_hbm, dst_hbm, vals_hbm, out_hbm, sidx, didx, rows, zbuf, acc, sem):
        cid = lax.axis_index("c")
        sid = lax.axis_index("s")
        wid = sid * NC + cid

        @pl.loop(0, NPT)
        def _(i):
            zbuf[i, :] = jnp.zeros((H,), jnp.float32)

        pltpu.sync_copy(zbuf, acc.at[pl.ds(sid * NPT, NPT)])
        plsc.subcore_barrier()

        @pl.loop(0, KMAX)
        def _(kk):
            c0 = wid + kk * NW

            @pl.when(c0 < NCHUNKS)
            def _():
                base = c0 * CHUNK
                pltpu.sync_copy(src_hbm.at[pl.ds(base, CHUNK)], sidx)
                pltpu.sync_copy(dst_hbm.at[pl.ds(base, CHUNK)], didx)
                pltpu.async_copy(vals_hbm.at[sidx], rows, sem).wait()
                pltpu.sync_copy(rows, acc.at[didx], add=True)

        plsc.subcore_barrier()
        pltpu.sync_copy(
            acc.at[pl.ds(sid * NPT, NPT)],
            out_hbm.at[cid, pl.ds(sid * NPT, NPT)],
        )

    return k(src, dst, vals)


def _tc_matmul1(x, W1):
    def body(x_ref, w_ref, o_ref):
        o_ref[...] = jnp.dot(
            x_ref[...], w_ref[...], preferred_element_type=jnp.float32
        )

    return pl.pallas_call(
        body, out_shape=jax.ShapeDtypeStruct((N, H), jnp.float32)
    )(x, W1)


def _tc_scale(h, degp_t):
    def body(h_ref, d_ref, hs_ref, dis_ref):
        deg = d_ref[:N, 0:1] + d_ref[:N, 1:2] + 1.0
        dis = lax.rsqrt(deg)
        dis_ref[...] = dis
        hs_ref[...] = h_ref[...] * dis

    return pl.pallas_call(
        body,
        out_shape=(
            jax.ShapeDtypeStruct((N, H), jnp.float32),
            jax.ShapeDtypeStruct((N, 1), jnp.float32),
        ),
    )(h, degp_t)


def _tc_mid(acc, hs, dis, W2, b1):
    def body(a_ref, hs_ref, dis_ref, w_ref, b_ref, o_ref):
        s = a_ref[0, :N, :] + a_ref[1, :N, :] + hs_ref[...]
        out1 = jnp.maximum(s * dis_ref[...] + b_ref[...], 0.0)
        o_ref[...] = (
            jnp.dot(out1, w_ref[...], preferred_element_type=jnp.float32)
            * dis_ref[...]
        )

    return pl.pallas_call(
        body, out_shape=jax.ShapeDtypeStruct((N, C), jnp.float32)
    )(acc, hs, dis, W2, b1)


def _tc_final(acc, h2s, dis, b2):
    def body(a_ref, hs_ref, dis_ref, b_ref, o_ref):
        s = a_ref[0, :N, :] + a_ref[1, :N, :] + hs_ref[...]
        o = s * dis_ref[...] + b_ref[...]
        m = jnp.max(o, axis=1, keepdims=True)
        lse = jnp.log(jnp.sum(jnp.exp(o - m), axis=1, keepdims=True)) + m
        o_ref[...] = o - lse

    return pl.pallas_call(
        body, out_shape=jax.ShapeDtypeStruct((N, C), jnp.float32)
    )(acc, h2s, dis, b2)


def kernel(x, edge_index, W1, b1, W2, b2):
    src = edge_index[0].astype(jnp.int32)
    dst = edge_index[1].astype(jnp.int32)
    degp = _sc_degree(dst)
    h = _tc_matmul1(x, W1)
    hs, dis = _tc_scale(h, degp.T)
    acc1 = _sc_scatter(src, dst, hs)
    h2s = _tc_mid(acc1, hs, dis, W2, b1.reshape(1, H))
    acc2 = _sc_scatter(src, dst, h2s)
    return _tc_final(acc2, h2s, dis, b2.reshape(1, C))


# speedup vs baseline: 58.6590x; 2.4970x over previous
import dataclasses
import functools

import jax
import jax.numpy as jnp
from jax import lax
from jax.experimental import pallas as pl
from jax.experimental.pallas import tpu as pltpu
from jax.experimental.pallas import tpu_sc as plsc

N = 10000
E = 320000
D = 128
H = 16
C = 16

NC = 2
NS = 16
NW = NC * NS
L = 16

NPAD = 10240
NPT = NPAD // NS

CHUNK = 128
KROWS = 2560
KPW = KROWS // NW
NB = 4
PADE = KROWS * CHUNK - E

DCH = 2000
EPW = E // NW

_MESH = dict(core_axis_name="c", subcore_axis_name="s")

_SC_PARAMS = pltpu.CompilerParams()
if "needs_layout_passes" in pltpu.CompilerParams.__dataclass_fields__:
    _SC_PARAMS = dataclasses.replace(
        _SC_PARAMS, needs_layout_passes=False, use_tc_tiling_on_sc=False
    )


def _sc_degree(dst):
    @functools.partial(
        pl.kernel,
        out_type=jax.ShapeDtypeStruct((NC, NPAD), jnp.float32),
        mesh=plsc.VectorSubcoreMesh(**_MESH),
        compiler_params=_SC_PARAMS,
        scratch_types=[
            pltpu.VMEM((NPAD,), jnp.float32),
            pltpu.VMEM((DCH,), jnp.int32),
            pltpu.VMEM((NS, NPT), jnp.float32),
            pltpu.VMEM((NPT,), jnp.float32),
            pltpu.VMEM_SHARED((NS, NPAD), jnp.float32),
        ],
    )
    def k(dst_hbm, out_hbm, hist, dbuf, redbuf, redout, stage):
        cid = lax.axis_index("c")
        sid = lax.axis_index("s")
        wid = sid * NC + cid

        @pl.loop(0, NPAD // L)
        def _(i):
            hist[pl.ds(i * L, L)] = jnp.zeros((L,), jnp.float32)

        base = wid * EPW

        @pl.loop(0, EPW // DCH)
        def _(c0):
            pltpu.sync_copy(dst_hbm.at[pl.ds(base + c0 * DCH, DCH)], dbuf)

            @pl.loop(0, DCH // L)
            def _(j):
                idx = dbuf[pl.ds(j * L, L)]
                plsc.addupdate_scatter(hist, [idx], jnp.ones((L,), jnp.float32))

        pltpu.sync_copy(hist, stage.at[sid])
        plsc.subcore_barrier()
        for r in range(NS):
            pltpu.sync_copy(stage.at[r, pl.ds(sid * NPT, NPT)], redbuf.at[r])

        @pl.loop(0, NPT // L)
        def _(i):
            v = redbuf[0, pl.ds(i * L, L)]
            for r in range(1, NS):
                v = v + redbuf[r, pl.ds(i * L, L)]
            redout[pl.ds(i * L, L)] = v

        pltpu.sync_copy(redout, out_hbm.at[cid, pl.ds(sid * NPT, NPT)])

    return k(dst)


def _sc_scatter(src2, dst2, vals):
    @functools.partial(
        pl.kernel,
        out_type=jax.ShapeDtypeStruct((NC, NPAD, H), jnp.float32),
        mesh=plsc.VectorSubcoreMesh(**_MESH),
        compiler_params=_SC_PARAMS,
        scratch_types=[
            pltpu.VMEM((KPW, CHUNK), jnp.int32),
            pltpu.VMEM((KPW, CHUNK), jnp.int32),
            pltpu.VMEM((NB, CHUNK, H), jnp.float32),
            pltpu.VMEM((NPT, H), jnp.float32),
            pltpu.VMEM_SHARED((NPAD, H), jnp.float32),
            pltpu.SemaphoreType.DMA,
        ] + [pltpu.SemaphoreType.DMA] * NB,
    )
    def k(src_hbm, dst_hbm, vals_hbm, out_hbm, sidx, didx, rows, zbuf, acc,
          sem0, *gsems):
        cid = lax.axis_index("c")
        sid = lax.axis_index("s")
        wid = sid * NC + cid
        wrow = wid * KPW

        pltpu.async_copy(src_hbm.at[pl.ds(wrow, KPW)], sidx, sem0)
        pltpu.async_copy(dst_hbm.at[pl.ds(wrow, KPW)], didx, sem0)

        @pl.loop(0, NPT)
        def _(i):
            zbuf[i, :] = jnp.zeros((H,), jnp.float32)

        pltpu.sync_copy(zbuf, acc.at[pl.ds(sid * NPT, NPT)])

        pltpu.make_async_copy(src_hbm.at[pl.ds(wrow, KPW)], sidx, sem0).wait()
        pltpu.make_async_copy(dst_hbm.at[pl.ds(wrow, KPW)], didx, sem0).wait()
        plsc.subcore_barrier()

        def gather(k_, b):
            pltpu.async_copy(vals_hbm.at[sidx.at[k_]], rows.at[b], gsems[b])

        def gwait(b):
            pltpu.make_async_copy(
                vals_hbm.at[sidx.at[0]], rows.at[b], gsems[b]
            ).wait()

        for b in range(NB):
            gather(b, b)

        @pl.loop(0, KPW // NB)
        def _(g):
            for b in range(NB):
                k_ = g * NB + b
                gwait(b)
                pltpu.sync_copy(rows.at[b], acc.at[didx.at[k_]], add=True)

                @pl.when(k_ + NB < KPW)
                def _():
                    gather(k_ + NB, b)

        plsc.subcore_barrier()
        pltpu.sync_copy(
            acc.at[pl.ds(sid * NPT, NPT)],
            out_hbm.at[cid, pl.ds(sid * NPT, NPT)],
        )

    return k(src2, dst2, vals)


def _tc_prep(x, W1, degp_t):
    def body(x_ref, w_ref, d_ref, hs_ref, dis_ref):
        deg = d_ref[:N, 0:1] + d_ref[:N, 1:2] + 1.0
        dis = lax.rsqrt(deg)
        dis_ref[...] = dis
        h = jnp.dot(x_ref[...], w_ref[...], preferred_element_type=jnp.float32)
        hs_ref[...] = h * dis

    return pl.pallas_call(
        body,
        out_shape=(
            jax.ShapeDtypeStruct((N, H), jnp.float32),
            jax.ShapeDtypeStruct((N, 1), jnp.float32),
        ),
    )(x, W1, degp_t)


def _tc_mid(acc, hs, dis, W2, b1):
    def body(a_ref, hs_ref, dis_ref, w_ref, b_ref, o_ref):
        s = a_ref[0, :N, :] + a_ref[1, :N, :] + hs_ref[...]
        out1 = jnp.maximum(s * dis_ref[...] + b_ref[...], 0.0)
        o_ref[...] = (
            jnp.dot(out1, w_ref[...], preferred_element_type=jnp.float32)
            * dis_ref[...]
        )

    return pl.pallas_call(
        body, out_shape=jax.ShapeDtypeStruct((N, C), jnp.float32)
    )(acc, hs, dis, W2, b1)


def _tc_final(acc, h2s, dis, b2):
    def body(a_ref, hs_ref, dis_ref, b_ref, o_ref):
        s = a_ref[0, :N, :] + a_ref[1, :N, :] + hs_ref[...]
        o = s * dis_ref[...] + b_ref[...]
        m = jnp.max(o, axis=1, keepdims=True)
        lse = jnp.log(jnp.sum(jnp.exp(o - m), axis=1, keepdims=True)) + m
        o_ref[...] = o - lse

    return pl.pallas_call(
        body, out_shape=jax.ShapeDtypeStruct((N, C), jnp.float32)
    )(acc, h2s, dis, b2)


def kernel(x, edge_index, W1, b1, W2, b2):
    src = edge_index[0].astype(jnp.int32)
    dst = edge_index[1].astype(jnp.int32)
    pad = jnp.arange(PADE, dtype=jnp.int32)
    src2 = jnp.concatenate([src, pad % N]).reshape(KROWS, CHUNK)
    dst2 = jnp.concatenate([dst, N + pad % (NPAD - N)]).reshape(KROWS, CHUNK)
    degp = _sc_degree(dst)
    hs, dis = _tc_prep(x, W1, degp.T)
    acc1 = _sc_scatter(src2, dst2, hs)
    h2s = _tc_mid(acc1, hs, dis, W2, b1.reshape(1, H))
    acc2 = _sc_scatter(src2, dst2, h2s)
    return _tc_final(acc2, h2s, dis, b2.reshape(1, C))
